# baseline (speedup 1.0000x reference)
import dataclasses
import functools

import jax
import jax.numpy as jnp
import numpy as np
from jax import lax
from jax.experimental import pallas as pl
from jax.experimental.pallas import tpu as pltpu
from jax.experimental.pallas import tpu_sc as plsc

N_NODES = 10000
D = 128
NC = 2
NS = 16
NW = NC * NS
UNIT = 128
LANES = 16
CHUNK_ROWS = 80
N_CHUNKS = N_NODES // CHUNK_ROWS
NI = 4
NB = 2

_PERM = np.concatenate(
    [np.arange(j * 32, (j + 1) * 32).reshape(16, 2).T.reshape(32)
     for j in range(D // 32)])


def _spmm_partials(Hb, row, col, val, units_per_tile):
    n = units_per_tile
    mesh = plsc.VectorSubcoreMesh(core_axis_name="c", subcore_axis_name="s")
    cp = pltpu.CompilerParams()
    if "needs_layout_passes" in pltpu.CompilerParams.__dataclass_fields__:
        cp = dataclasses.replace(cp, needs_layout_passes=False)
    if "use_tc_tiling_on_sc" in pltpu.CompilerParams.__dataclass_fields__:
        cp = dataclasses.replace(cp, use_tc_tiling_on_sc=False)

    scratch = (
        [pltpu.VMEM((UNIT,), jnp.int32)] * NI
        + [pltpu.VMEM((UNIT,), jnp.int32)] * NI
        + [pltpu.VMEM((UNIT,), jnp.float32)] * NI
        + [pltpu.VMEM((UNIT,), jnp.int32)] * NB
        + [pltpu.VMEM((UNIT, D // 2), jnp.int32)] * NB
        + [pltpu.VMEM((UNIT, D), jnp.float32)] * NB
        + [pltpu.VMEM_SHARED((N_NODES, D), jnp.float32)]
        + [pltpu.SemaphoreType.DMA] * (NI + 2 * NB + 1)
    )

    @functools.partial(
        pl.kernel,
        mesh=mesh,
        compiler_params=cp,
        out_type=jax.ShapeDtypeStruct((NC, N_NODES, D), jnp.float32),
        scratch_types=scratch,
    )
    def spmm(h_hbm, row_hbm, col_hbm, val_hbm, out_hbm, *sc):
        colv = sc[0:NI]
        rowv = sc[NI:2 * NI]
        valv = sc[2 * NI:3 * NI]
        rowS = sc[3 * NI:3 * NI + NB]
        gbuf = sc[3 * NI + NB:3 * NI + 2 * NB]
        fbuf = sc[3 * NI + 2 * NB:3 * NI + 3 * NB]
        acc = sc[3 * NI + 3 * NB]
        base_s = 3 * NI + 3 * NB + 1
        sem_i = sc[base_s:base_s + NI]
        sem_g = sc[base_s + NI:base_s + NI + NB]
        sem_s = sc[base_s + NI + NB:base_s + NI + 2 * NB]
        sem_z = sc[base_s + NI + 2 * NB]

        c = lax.axis_index("c")
        s = lax.axis_index("s")
        wid = s * NC + c

        def issue_idx(uu, i):
            base = (wid * n + uu) * UNIT
            pltpu.async_copy(col_hbm.at[pl.ds(base, UNIT)], colv[i], sem_i[i])
            pltpu.async_copy(row_hbm.at[pl.ds(base, UNIT)], rowv[i], sem_i[i])
            pltpu.async_copy(val_hbm.at[pl.ds(base, UNIT)], valv[i], sem_i[i])

        def wait_idx(i):
            src = col_hbm.at[pl.ds(0, UNIT)]
            pltpu.make_async_copy(src, colv[i], sem_i[i]).wait()
            pltpu.make_async_copy(src, rowv[i], sem_i[i]).wait()
            vsrc = val_hbm.at[pl.ds(0, UNIT)]
            pltpu.make_async_copy(vsrc, valv[i], sem_i[i]).wait()

        def issue_gather(i, g):
            pltpu.async_copy(h_hbm.at[colv[i]], gbuf[g], sem_g[g])

        def wait_gather(g):
            pltpu.make_async_copy(h_hbm.at[colv[0]], gbuf[g],
                                  sem_g[g]).wait()

        def issue_scatter(i, f):
            for j in range(UNIT // LANES):
                sl = pl.ds(j * LANES, LANES)
                rowS[f][sl] = rowv[i][sl]
            pltpu.async_copy(fbuf[f], acc.at[rowS[f]], sem_s[f], add=True)

        def wait_scatter(f):
            pltpu.make_async_copy(fbuf[f], acc.at[rowS[f]], sem_s[f]).wait()

        def scale(i, g):
            @pl.loop(0, UNIT)
            def _(e):
                vv = plsc.load_gather(
                    valv[i], [jnp.zeros((LANES,), jnp.int32) + e])
                for j in range(D // 32):
                    xi = gbuf[g][e, pl.ds(j * LANES, LANES)]
                    x = plsc.bitcast(xi, jnp.bfloat16)
                    lo, hi = plsc.unpack(x, format=plsc.PackFormat.INTERLEAVED)
                    fbuf[g][e, pl.ds(j * 32, LANES)] = lo * vv
                    fbuf[g][e, pl.ds(j * 32 + LANES, LANES)] = hi * vv

        issue_idx(0, 0)
        issue_idx(1, 1)

        zeros16 = jnp.zeros((LANES,), jnp.float32)

        @pl.loop(0, CHUNK_ROWS)
        def _(e):
            for j in range(D // LANES):
                fbuf[1][e, pl.ds(j * LANES, LANES)] = zeros16

        zbuf = fbuf[1].at[pl.ds(0, CHUNK_ROWS)]

        @pl.loop(0, (N_CHUNKS + NS - 1) // NS)
        def _(i):
            cid = s + i * NS

            @pl.when(cid < N_CHUNKS)
            def _():
                pltpu.async_copy(
                    zbuf, acc.at[pl.ds(cid * CHUNK_ROWS, CHUNK_ROWS)], sem_z)

        wait_idx(0)
        issue_gather(0, 0)

        @pl.loop(0, (N_CHUNKS + NS - 1) // NS)
        def _(i):
            cid = s + i * NS

            @pl.when(cid < N_CHUNKS)
            def _():
                pltpu.make_async_copy(
                    zbuf, acc.at[pl.ds(cid * CHUNK_ROWS, CHUNK_ROWS)],
                    sem_z).wait()

        plsc.subcore_barrier()

        @pl.loop(0, n // NI)
        def _(u):
            for B in range(NI):
                uu = u * NI + B
                Bi = B % NI
                Bb = B % NB
                Bn = (B + 1) % NB

                @pl.when(uu >= 2)
                def _():
                    wait_scatter(Bb)

                wait_gather(Bb)

                @pl.when(uu + 1 < n)
                def _():
                    wait_idx((Bi + 1) % NI)
                    issue_gather((Bi + 1) % NI, Bn)

                scale(Bi, Bb)
                issue_scatter(Bi, Bb)

                @pl.when(uu + 2 < n)
                def _():
                    issue_idx(uu + 2, (Bi + 2) % NI)

        wait_scatter(0)
        wait_scatter(1)
        plsc.subcore_barrier()

        @pl.loop(0, (N_CHUNKS + NS - 1) // NS)
        def _(i):
            cid = s + i * NS

            @pl.when(cid < N_CHUNKS)
            def _():
                pltpu.async_copy(
                    acc.at[pl.ds(cid * CHUNK_ROWS, CHUNK_ROWS)],
                    out_hbm.at[c, pl.ds(cid * CHUNK_ROWS, CHUNK_ROWS)], sem_z)

        @pl.loop(0, (N_CHUNKS + NS - 1) // NS)
        def _(i):
            cid = s + i * NS

            @pl.when(cid < N_CHUNKS)
            def _():
                pltpu.make_async_copy(
                    acc.at[pl.ds(cid * CHUNK_ROWS, CHUNK_ROWS)],
                    out_hbm.at[c, pl.ds(cid * CHUNK_ROWS, CHUNK_ROWS)],
                    sem_z).wait()

    return spmm(Hb, row, col, val)


def _finish_body(p_ref, w_ref, b_ref, o_ref):
    x = p_ref[0] + p_ref[1]
    y = jnp.dot(x, w_ref[...], preferred_element_type=jnp.float32)
    o_ref[...] = jnp.maximum(y + b_ref[...], 0.0)


def _finish(partials, Wp, bias):
    BM = 1000
    return pl.pallas_call(
        _finish_body,
        grid=(N_NODES // BM,),
        in_specs=[
            pl.BlockSpec((NC, BM, D), lambda i: (0, i, 0)),
            pl.BlockSpec((D, D), lambda i: (0, 0)),
            pl.BlockSpec((1, D), lambda i: (0, 0)),
        ],
        out_specs=pl.BlockSpec((BM, D), lambda i: (i, 0)),
        out_shape=jax.ShapeDtypeStruct((N_NODES, D), jnp.float32),
    )(partials, Wp, bias.reshape(1, D))


def kernel(A_edge_index, A_values, H, W, bias):
    row = A_edge_index[0]
    col = A_edge_index[1]
    E = row.shape[0]
    grain = NW * UNIT * NI
    E_pad = ((E + grain - 1) // grain) * grain
    pad = E_pad - E
    if pad:
        spread = (jnp.arange(pad, dtype=jnp.int32) * 13) % N_NODES
        row = jnp.concatenate([row, spread])
        col = jnp.concatenate([col, spread])
        val = jnp.concatenate([A_values, jnp.zeros((pad,), jnp.float32)])
    else:
        val = A_values
    units_per_tile = E_pad // (NW * UNIT)

    Hb = H.astype(jnp.bfloat16)
    H32 = jax.lax.bitcast_convert_type(
        Hb.reshape(N_NODES, D // 2, 2), jnp.int32)
    Wp = W[jnp.asarray(_PERM), :]
    partials = _spmm_partials(H32, row, col, val, units_per_tile)
    return _finish(partials, Wp, bias)


# speedup vs baseline: 2.0576x; 2.0576x over previous
import dataclasses
import functools

import jax
import jax.numpy as jnp
from jax import lax
from jax.experimental import pallas as pl
from jax.experimental.pallas import tpu as pltpu
from jax.experimental.pallas import tpu_sc as plsc

N_NODES = 10000
D = 128
NC = 2
NS = 16
NW = NC * NS
UNIT = 128
SLAB = 3 * UNIT
LANES = 16
CHUNK_ROWS = 80
N_CHUNKS = N_NODES // CHUNK_ROWS
DEPTH = 3


def _spmm_partials(H, packed, units_per_tile):
    n = units_per_tile
    mesh = plsc.VectorSubcoreMesh(core_axis_name="c", subcore_axis_name="s")
    cp = pltpu.CompilerParams()
    if "needs_layout_passes" in pltpu.CompilerParams.__dataclass_fields__:
        cp = dataclasses.replace(cp, needs_layout_passes=False)

    scratch = (
        [pltpu.VMEM((SLAB,), jnp.int32)] * DEPTH
        + [pltpu.VMEM((UNIT,), jnp.int32)] * DEPTH
        + [pltpu.VMEM((UNIT, D), jnp.float32)] * DEPTH
        + [pltpu.VMEM_SHARED((N_NODES, D), jnp.float32)]
        + [pltpu.SemaphoreType.DMA] * (3 * DEPTH + 1)
    )

    @functools.partial(
        pl.kernel,
        mesh=mesh,
        compiler_params=cp,
        out_type=jax.ShapeDtypeStruct((NC, N_NODES, D), jnp.float32),
        scratch_types=scratch,
    )
    def spmm(h_hbm, pk_hbm, out_hbm, *sc):
        pidx = sc[0:DEPTH]
        rowS = sc[DEPTH:2 * DEPTH]
        msgs = sc[2 * DEPTH:3 * DEPTH]
        acc = sc[3 * DEPTH]
        sem_i = sc[3 * DEPTH + 1:3 * DEPTH + 1 + DEPTH]
        sem_g = sc[3 * DEPTH + 1 + DEPTH:3 * DEPTH + 1 + 2 * DEPTH]
        sem_s = sc[3 * DEPTH + 1 + 2 * DEPTH:3 * DEPTH + 1 + 3 * DEPTH]
        sem_z = sc[3 * DEPTH + 1 + 3 * DEPTH]

        c = lax.axis_index("c")
        s = lax.axis_index("s")
        wid = s * NC + c

        def issue_idx(uu, b):
            base = (wid * n + uu) * SLAB
            pltpu.async_copy(pk_hbm.at[pl.ds(base, SLAB)], pidx[b], sem_i[b])

        def wait_idx(b):
            pltpu.make_async_copy(pk_hbm.at[pl.ds(0, SLAB)], pidx[b],
                                  sem_i[b]).wait()

        def issue_gather(b):
            pltpu.async_copy(h_hbm.at[pidx[b].at[pl.ds(0, UNIT)]], msgs[b],
                             sem_g[b])

        def wait_gather(b):
            pltpu.make_async_copy(h_hbm.at[pidx[b].at[pl.ds(0, UNIT)]],
                                  msgs[b], sem_g[b]).wait()

        def issue_scatter(b):
            for j in range(UNIT // LANES):
                pidx_sl = pl.ds(UNIT + j * LANES, LANES)
                rowS[b][pl.ds(j * LANES, LANES)] = pidx[b][pidx_sl]
            pltpu.async_copy(msgs[b], acc.at[rowS[b]], sem_s[b], add=True)

        def wait_scatter(b):
            pltpu.make_async_copy(msgs[b], acc.at[rowS[b]], sem_s[b]).wait()

        def scale(b):
            @pl.loop(0, UNIT)
            def _(e):
                vv_bits = plsc.load_gather(
                    pidx[b], [jnp.zeros((LANES,), jnp.int32) + (2 * UNIT + e)])
                vv = plsc.bitcast(vv_bits, jnp.float32)
                for j in range(D // LANES):
                    sl = pl.ds(j * LANES, LANES)
                    msgs[b][e, sl] = msgs[b][e, sl] * vv

        for b in range(DEPTH):
            issue_idx(b, b)

        zeros16 = jnp.zeros((LANES,), jnp.float32)

        @pl.loop(0, CHUNK_ROWS)
        def _(e):
            for j in range(D // LANES):
                msgs[DEPTH - 1][e, pl.ds(j * LANES, LANES)] = zeros16

        zbuf = msgs[DEPTH - 1].at[pl.ds(0, CHUNK_ROWS)]

        @pl.loop(0, (N_CHUNKS + NS - 1) // NS)
        def _(i):
            cid = s + i * NS

            @pl.when(cid < N_CHUNKS)
            def _():
                pltpu.async_copy(
                    zbuf, acc.at[pl.ds(cid * CHUNK_ROWS, CHUNK_ROWS)], sem_z)

        for b in range(DEPTH - 1):
            wait_idx(b)
            issue_gather(b)

        @pl.loop(0, (N_CHUNKS + NS - 1) // NS)
        def _(i):
            cid = s + i * NS

            @pl.when(cid < N_CHUNKS)
            def _():
                pltpu.make_async_copy(
                    zbuf, acc.at[pl.ds(cid * CHUNK_ROWS, CHUNK_ROWS)],
                    sem_z).wait()

        plsc.subcore_barrier()

        @pl.loop(0, n // DEPTH)
        def _(u):
            for b in range(DEPTH):
                uu = u * DEPTH + b
                bN = (b + DEPTH - 1) % DEPTH
                wait_gather(b)
                scale(b)
                issue_scatter(b)

                @pl.when(uu + DEPTH < n)
                def _():
                    issue_idx(uu + DEPTH, b)

                @pl.when(uu >= 1)
                def _():
                    wait_scatter(bN)

                @pl.when(uu + DEPTH - 1 < n)
                def _():
                    wait_idx(bN)
                    issue_gather(bN)

        wait_scatter((n - 1) % DEPTH)
        plsc.subcore_barrier()

        @pl.loop(0, (N_CHUNKS + NS - 1) // NS)
        def _(i):
            cid = s + i * NS

            @pl.when(cid < N_CHUNKS)
            def _():
                pltpu.async_copy(
                    acc.at[pl.ds(cid * CHUNK_ROWS, CHUNK_ROWS)],
                    out_hbm.at[c, pl.ds(cid * CHUNK_ROWS, CHUNK_ROWS)], sem_z)

        @pl.loop(0, (N_CHUNKS + NS - 1) // NS)
        def _(i):
            cid = s + i * NS

            @pl.when(cid < N_CHUNKS)
            def _():
                pltpu.make_async_copy(
                    acc.at[pl.ds(cid * CHUNK_ROWS, CHUNK_ROWS)],
                    out_hbm.at[c, pl.ds(cid * CHUNK_ROWS, CHUNK_ROWS)],
                    sem_z).wait()

    return spmm(H, packed)


def _finish_body(p_ref, w_ref, b_ref, o_ref):
    x = p_ref[0] + p_ref[1]
    y = jnp.dot(x, w_ref[...], preferred_element_type=jnp.float32)
    o_ref[...] = jnp.maximum(y + b_ref[...], 0.0)


def _finish(partials, W, bias):
    BM = 1000
    return pl.pallas_call(
        _finish_body,
        grid=(N_NODES // BM,),
        in_specs=[
            pl.BlockSpec((NC, BM, D), lambda i: (0, i, 0)),
            pl.BlockSpec((D, D), lambda i: (0, 0)),
            pl.BlockSpec((1, D), lambda i: (0, 0)),
        ],
        out_specs=pl.BlockSpec((BM, D), lambda i: (i, 0)),
        out_shape=jax.ShapeDtypeStruct((N_NODES, D), jnp.float32),
    )(partials, W, bias.reshape(1, D))


def kernel(A_edge_index, A_values, H, W, bias):
    row = A_edge_index[0]
    col = A_edge_index[1]
    E = row.shape[0]
    grain = NW * UNIT * DEPTH
    E_pad = ((E + grain - 1) // grain) * grain
    pad = E_pad - E
    if pad:
        spread = (jnp.arange(pad, dtype=jnp.int32) * 13) % N_NODES
        row = jnp.concatenate([row, spread])
        col = jnp.concatenate([col, spread])
        val = jnp.concatenate([A_values, jnp.zeros((pad,), jnp.float32)])
    else:
        val = A_values
    units_per_tile = E_pad // (NW * UNIT)

    vbits = jax.lax.bitcast_convert_type(val, jnp.int32)
    packed = jnp.stack(
        [col.reshape(-1, UNIT), row.reshape(-1, UNIT),
         vbits.reshape(-1, UNIT)], axis=1).reshape(-1)

    partials = _spmm_partials(H, packed, units_per_tile)
    return _finish(partials, W, bias)


# speedup vs baseline: 2.1208x; 1.0307x over previous
import dataclasses
import functools

import jax
import jax.numpy as jnp
from jax import lax
from jax.experimental import pallas as pl
from jax.experimental.pallas import tpu as pltpu
from jax.experimental.pallas import tpu_sc as plsc

N_NODES = 10000
D = 128
NC = 2
NS = 16
NW = NC * NS
UNIT = 128
LANES = 16
CHUNK_ROWS = 80
N_CHUNKS = N_NODES // CHUNK_ROWS


DEPTH = 3


def _spmm_partials(HW, row, col, val, units_per_tile):
    n = units_per_tile
    mesh = plsc.VectorSubcoreMesh(core_axis_name="c", subcore_axis_name="s")
    cp = pltpu.CompilerParams()
    if "needs_layout_passes" in pltpu.CompilerParams.__dataclass_fields__:
        cp = dataclasses.replace(cp, needs_layout_passes=False)

    scratch = (
        [pltpu.VMEM((UNIT,), jnp.int32)] * DEPTH
        + [pltpu.VMEM((UNIT,), jnp.int32)] * DEPTH
        + [pltpu.VMEM((UNIT,), jnp.float32)] * DEPTH
        + [pltpu.VMEM((UNIT,), jnp.int32)] * DEPTH
        + [pltpu.VMEM((UNIT, D), jnp.float32)] * DEPTH
        + [pltpu.VMEM_SHARED((N_NODES, D), jnp.float32)]
        + [pltpu.SemaphoreType.DMA] * (3 * DEPTH + 1)
    )

    @functools.partial(
        pl.kernel,
        mesh=mesh,
        compiler_params=cp,
        out_type=jax.ShapeDtypeStruct((NC, N_NODES, D), jnp.float32),
        scratch_types=scratch,
    )
    def spmm(hw_hbm, row_hbm, col_hbm, val_hbm, out_hbm, *sc):
        colv = sc[0:DEPTH]
        rowv = sc[DEPTH:2 * DEPTH]
        valv = sc[2 * DEPTH:3 * DEPTH]
        rowS = sc[3 * DEPTH:4 * DEPTH]
        msgs = sc[4 * DEPTH:5 * DEPTH]
        acc = sc[5 * DEPTH]
        sem_i = sc[5 * DEPTH + 1:5 * DEPTH + 1 + DEPTH]
        sem_g = sc[5 * DEPTH + 1 + DEPTH:5 * DEPTH + 1 + 2 * DEPTH]
        sem_s = sc[5 * DEPTH + 1 + 2 * DEPTH:5 * DEPTH + 1 + 3 * DEPTH]
        sem_z = sc[5 * DEPTH + 1 + 3 * DEPTH]

        c = lax.axis_index("c")
        s = lax.axis_index("s")
        wid = s * NC + c

        def issue_idx(uu, b):
            base = (wid * n + uu) * UNIT
            pltpu.async_copy(col_hbm.at[pl.ds(base, UNIT)], colv[b], sem_i[b])
            pltpu.async_copy(row_hbm.at[pl.ds(base, UNIT)], rowv[b], sem_i[b])
            pltpu.async_copy(val_hbm.at[pl.ds(base, UNIT)], valv[b], sem_i[b])

        def wait_idx(b):
            src = col_hbm.at[pl.ds(0, UNIT)]
            pltpu.make_async_copy(src, colv[b], sem_i[b]).wait()
            pltpu.make_async_copy(src, rowv[b], sem_i[b]).wait()
            vsrc = val_hbm.at[pl.ds(0, UNIT)]
            pltpu.make_async_copy(vsrc, valv[b], sem_i[b]).wait()

        def issue_gather(b):
            pltpu.async_copy(hw_hbm.at[colv[b]], msgs[b], sem_g[b])

        def wait_gather(b):
            pltpu.make_async_copy(hw_hbm.at[colv[b]], msgs[b],
                                  sem_g[b]).wait()

        def issue_scatter(b):
            for j in range(UNIT // LANES):
                sl = pl.ds(j * LANES, LANES)
                rowS[b][sl] = rowv[b][sl]
            pltpu.async_copy(msgs[b], acc.at[rowS[b]], sem_s[b], add=True)

        def wait_scatter(b):
            pltpu.make_async_copy(msgs[b], acc.at[rowS[b]], sem_s[b]).wait()

        def scale(b):
            @pl.loop(0, UNIT)
            def _(e):
                vv = plsc.load_gather(
                    valv[b], [jnp.zeros((LANES,), jnp.int32) + e])
                for j in range(D // LANES):
                    sl = pl.ds(j * LANES, LANES)
                    msgs[b][e, sl] = msgs[b][e, sl] * vv

        for b in range(DEPTH):
            issue_idx(b, b)

        zeros16 = jnp.zeros((LANES,), jnp.float32)

        @pl.loop(0, CHUNK_ROWS)
        def _(e):
            for j in range(D // LANES):
                msgs[DEPTH - 1][e, pl.ds(j * LANES, LANES)] = zeros16

        zbuf = msgs[DEPTH - 1].at[pl.ds(0, CHUNK_ROWS)]

        @pl.loop(0, (N_CHUNKS + NS - 1) // NS)
        def _(i):
            cid = s + i * NS

            @pl.when(cid < N_CHUNKS)
            def _():
                pltpu.async_copy(
                    zbuf, acc.at[pl.ds(cid * CHUNK_ROWS, CHUNK_ROWS)], sem_z)

        for b in range(DEPTH - 1):
            wait_idx(b)
            issue_gather(b)

        @pl.loop(0, (N_CHUNKS + NS - 1) // NS)
        def _(i):
            cid = s + i * NS

            @pl.when(cid < N_CHUNKS)
            def _():
                pltpu.make_async_copy(
                    zbuf, acc.at[pl.ds(cid * CHUNK_ROWS, CHUNK_ROWS)],
                    sem_z).wait()

        plsc.subcore_barrier()

        @pl.loop(0, n // DEPTH)
        def _(u):
            for b in range(DEPTH):
                uu = u * DEPTH + b
                bN = (b + DEPTH - 1) % DEPTH
                wait_gather(b)
                scale(b)
                issue_scatter(b)

                @pl.when(uu + DEPTH < n)
                def _():
                    issue_idx(uu + DEPTH, b)

                @pl.when(uu >= 1)
                def _():
                    wait_scatter(bN)

                @pl.when(uu + DEPTH - 1 < n)
                def _():
                    wait_idx(bN)
                    issue_gather(bN)

        wait_scatter((n - 1) % DEPTH)
        plsc.subcore_barrier()

        @pl.loop(0, (N_CHUNKS + NS - 1) // NS)
        def _(i):
            cid = s + i * NS

            @pl.when(cid < N_CHUNKS)
            def _():
                pltpu.async_copy(
                    acc.at[pl.ds(cid * CHUNK_ROWS, CHUNK_ROWS)],
                    out_hbm.at[c, pl.ds(cid * CHUNK_ROWS, CHUNK_ROWS)], sem_z)

        @pl.loop(0, (N_CHUNKS + NS - 1) // NS)
        def _(i):
            cid = s + i * NS

            @pl.when(cid < N_CHUNKS)
            def _():
                pltpu.make_async_copy(
                    acc.at[pl.ds(cid * CHUNK_ROWS, CHUNK_ROWS)],
                    out_hbm.at[c, pl.ds(cid * CHUNK_ROWS, CHUNK_ROWS)],
                    sem_z).wait()

    return spmm(HW, row, col, val)


def _finish_body(p_ref, w_ref, b_ref, o_ref):
    x = p_ref[0] + p_ref[1]
    y = jnp.dot(x, w_ref[...], preferred_element_type=jnp.float32)
    o_ref[...] = jnp.maximum(y + b_ref[...], 0.0)


def _finish(partials, W, bias):
    BM = 1000
    return pl.pallas_call(
        _finish_body,
        grid=(N_NODES // BM,),
        in_specs=[
            pl.BlockSpec((NC, BM, D), lambda i: (0, i, 0)),
            pl.BlockSpec((D, D), lambda i: (0, 0)),
            pl.BlockSpec((1, D), lambda i: (0, 0)),
        ],
        out_specs=pl.BlockSpec((BM, D), lambda i: (i, 0)),
        out_shape=jax.ShapeDtypeStruct((N_NODES, D), jnp.float32),
    )(partials, W, bias.reshape(1, D))


def kernel(A_edge_index, A_values, H, W, bias):
    row = A_edge_index[0]
    col = A_edge_index[1]
    E = row.shape[0]
    grain = NW * UNIT * DEPTH
    E_pad = ((E + grain - 1) // grain) * grain
    pad = E_pad - E
    if pad:
        spread = (jnp.arange(pad, dtype=jnp.int32) * 13) % N_NODES
        row = jnp.concatenate([row, spread])
        col = jnp.concatenate([col, spread])
        val = jnp.concatenate([A_values, jnp.zeros((pad,), jnp.float32)])
    else:
        val = A_values
    units_per_tile = E_pad // (NW * UNIT)

    partials = _spmm_partials(H, row, col, val, units_per_tile)
    return _finish(partials, W, bias)


# speedup vs baseline: 2.2040x; 1.0392x over previous
import dataclasses
import functools

import jax
import jax.numpy as jnp
from jax import lax
from jax.experimental import pallas as pl
from jax.experimental.pallas import tpu as pltpu
from jax.experimental.pallas import tpu_sc as plsc

N_NODES = 10000
D = 128
NC = 2
NS = 16
NW = NC * NS
UNIT = 128
LANES = 16
CHUNK_ROWS = 80
N_CHUNKS = N_NODES // CHUNK_ROWS


DEPTH = 3


def _spmm_partials(HW, row, col, val, units_per_tile):
    n = units_per_tile
    mesh = plsc.VectorSubcoreMesh(core_axis_name="c", subcore_axis_name="s")
    cp = pltpu.CompilerParams()
    if "needs_layout_passes" in pltpu.CompilerParams.__dataclass_fields__:
        cp = dataclasses.replace(cp, needs_layout_passes=False)

    scratch = (
        [pltpu.VMEM((UNIT,), jnp.int32)] * DEPTH
        + [pltpu.VMEM((UNIT,), jnp.int32)] * DEPTH
        + [pltpu.VMEM((UNIT,), jnp.float32)] * DEPTH
        + [pltpu.VMEM((UNIT,), jnp.int32)] * DEPTH
        + [pltpu.VMEM((UNIT, D), jnp.float32)] * DEPTH
        + [pltpu.VMEM_SHARED((N_NODES, D), jnp.float32)]
        + [pltpu.SemaphoreType.DMA] * (3 * DEPTH + 1)
    )

    @functools.partial(
        pl.kernel,
        mesh=mesh,
        compiler_params=cp,
        out_type=jax.ShapeDtypeStruct((NC, N_NODES, D), jnp.float32),
        scratch_types=scratch,
    )
    def spmm(hw_hbm, row_hbm, col_hbm, val_hbm, out_hbm, *sc):
        colv = sc[0:DEPTH]
        rowv = sc[DEPTH:2 * DEPTH]
        valv = sc[2 * DEPTH:3 * DEPTH]
        rowS = sc[3 * DEPTH:4 * DEPTH]
        msgs = sc[4 * DEPTH:5 * DEPTH]
        acc = sc[5 * DEPTH]
        sem_i = sc[5 * DEPTH + 1:5 * DEPTH + 1 + DEPTH]
        sem_g = sc[5 * DEPTH + 1 + DEPTH:5 * DEPTH + 1 + 2 * DEPTH]
        sem_s = sc[5 * DEPTH + 1 + 2 * DEPTH:5 * DEPTH + 1 + 3 * DEPTH]
        sem_z = sc[5 * DEPTH + 1 + 3 * DEPTH]

        c = lax.axis_index("c")
        s = lax.axis_index("s")
        wid = s * NC + c

        def issue_idx(uu, b):
            base = (wid * n + uu) * UNIT
            pltpu.async_copy(col_hbm.at[pl.ds(base, UNIT)], colv[b], sem_i[b])
            pltpu.async_copy(row_hbm.at[pl.ds(base, UNIT)], rowv[b], sem_i[b])
            pltpu.async_copy(val_hbm.at[pl.ds(base, UNIT)], valv[b], sem_i[b])

        def wait_idx(b):
            src = col_hbm.at[pl.ds(0, UNIT)]
            pltpu.make_async_copy(src, colv[b], sem_i[b]).wait()
            pltpu.make_async_copy(src, rowv[b], sem_i[b]).wait()
            vsrc = val_hbm.at[pl.ds(0, UNIT)]
            pltpu.make_async_copy(vsrc, valv[b], sem_i[b]).wait()

        def issue_gather(b):
            pltpu.async_copy(hw_hbm.at[colv[b]], msgs[b], sem_g[b])

        def wait_gather(b):
            pltpu.make_async_copy(hw_hbm.at[colv[b]], msgs[b],
                                  sem_g[b]).wait()

        def issue_scatter(b):
            for j in range(UNIT // LANES):
                sl = pl.ds(j * LANES, LANES)
                rowS[b][sl] = rowv[b][sl]
            pltpu.async_copy(msgs[b], acc.at[rowS[b]], sem_s[b], add=True)

        def wait_scatter(b):
            pltpu.make_async_copy(msgs[b], acc.at[rowS[b]], sem_s[b]).wait()

        def scale(b):
            @pl.loop(0, UNIT, step=2)
            def _(e):
                vv0 = plsc.load_gather(
                    valv[b], [jnp.zeros((LANES,), jnp.int32) + e])
                vv1 = plsc.load_gather(
                    valv[b], [jnp.zeros((LANES,), jnp.int32) + (e + 1)])
                for j in range(D // LANES):
                    sl = pl.ds(j * LANES, LANES)
                    msgs[b][e, sl] = msgs[b][e, sl] * vv0
                    msgs[b][e + 1, sl] = msgs[b][e + 1, sl] * vv1

        for b in range(DEPTH):
            issue_idx(b, b)

        zeros16 = jnp.zeros((LANES,), jnp.float32)

        @pl.loop(0, CHUNK_ROWS)
        def _(e):
            for j in range(D // LANES):
                msgs[DEPTH - 1][e, pl.ds(j * LANES, LANES)] = zeros16

        zbuf = msgs[DEPTH - 1].at[pl.ds(0, CHUNK_ROWS)]

        @pl.loop(0, (N_CHUNKS + NS - 1) // NS)
        def _(i):
            cid = s + i * NS

            @pl.when(cid < N_CHUNKS)
            def _():
                pltpu.async_copy(
                    zbuf, acc.at[pl.ds(cid * CHUNK_ROWS, CHUNK_ROWS)], sem_z)

        for b in range(DEPTH - 1):
            wait_idx(b)
            issue_gather(b)

        @pl.loop(0, (N_CHUNKS + NS - 1) // NS)
        def _(i):
            cid = s + i * NS

            @pl.when(cid < N_CHUNKS)
            def _():
                pltpu.make_async_copy(
                    zbuf, acc.at[pl.ds(cid * CHUNK_ROWS, CHUNK_ROWS)],
                    sem_z).wait()

        plsc.subcore_barrier()

        @pl.loop(0, n // DEPTH)
        def _(u):
            for b in range(DEPTH):
                uu = u * DEPTH + b
                bN = (b + DEPTH - 1) % DEPTH
                wait_gather(b)
                scale(b)
                issue_scatter(b)

                @pl.when(uu + DEPTH < n)
                def _():
                    issue_idx(uu + DEPTH, b)

                @pl.when(uu >= 1)
                def _():
                    wait_scatter(bN)

                @pl.when(uu + DEPTH - 1 < n)
                def _():
                    wait_idx(bN)
                    issue_gather(bN)

        wait_scatter((n - 1) % DEPTH)
        plsc.subcore_barrier()

        @pl.loop(0, (N_CHUNKS + NS - 1) // NS)
        def _(i):
            cid = s + i * NS

            @pl.when(cid < N_CHUNKS)
            def _():
                pltpu.async_copy(
                    acc.at[pl.ds(cid * CHUNK_ROWS, CHUNK_ROWS)],
                    out_hbm.at[c, pl.ds(cid * CHUNK_ROWS, CHUNK_ROWS)], sem_z)

        @pl.loop(0, (N_CHUNKS + NS - 1) // NS)
        def _(i):
            cid = s + i * NS

            @pl.when(cid < N_CHUNKS)
            def _():
                pltpu.make_async_copy(
                    acc.at[pl.ds(cid * CHUNK_ROWS, CHUNK_ROWS)],
                    out_hbm.at[c, pl.ds(cid * CHUNK_ROWS, CHUNK_ROWS)],
                    sem_z).wait()

    return spmm(HW, row, col, val)


def _finish_body(p_ref, w_ref, b_ref, o_ref):
    x = p_ref[0] + p_ref[1]
    y = jnp.dot(x, w_ref[...], preferred_element_type=jnp.float32)
    o_ref[...] = jnp.maximum(y + b_ref[...], 0.0)


def _finish(partials, W, bias):
    BM = 1000
    return pl.pallas_call(
        _finish_body,
        grid=(N_NODES // BM,),
        in_specs=[
            pl.BlockSpec((NC, BM, D), lambda i: (0, i, 0)),
            pl.BlockSpec((D, D), lambda i: (0, 0)),
            pl.BlockSpec((1, D), lambda i: (0, 0)),
        ],
        out_specs=pl.BlockSpec((BM, D), lambda i: (i, 0)),
        out_shape=jax.ShapeDtypeStruct((N_NODES, D), jnp.float32),
    )(partials, W, bias.reshape(1, D))


def kernel(A_edge_index, A_values, H, W, bias):
    row = A_edge_index[0]
    col = A_edge_index[1]
    E = row.shape[0]
    grain = NW * UNIT * DEPTH
    E_pad = ((E + grain - 1) // grain) * grain
    pad = E_pad - E
    if pad:
        spread = (jnp.arange(pad, dtype=jnp.int32) * 13) % N_NODES
        row = jnp.concatenate([row, spread])
        col = jnp.concatenate([col, spread])
        val = jnp.concatenate([A_values, jnp.zeros((pad,), jnp.float32)])
    else:
        val = A_values
    units_per_tile = E_pad // (NW * UNIT)

    partials = _spmm_partials(H, row, col, val, units_per_tile)
    return _finish(partials, W, bias)


# speedup vs baseline: 2.2298x; 1.0117x over previous
import dataclasses
import functools

import jax
import jax.numpy as jnp
from jax import lax
from jax.experimental import pallas as pl
from jax.experimental.pallas import tpu as pltpu
from jax.experimental.pallas import tpu_sc as plsc

N_NODES = 10000
D = 128
NC = 2
NS = 16
NW = NC * NS
UNIT = 128
LANES = 16
CHUNK_ROWS = 80
N_CHUNKS = N_NODES // CHUNK_ROWS


DEPTH = 3


def _spmm_partials(HW, row, col, val, units_per_tile):
    n = units_per_tile
    mesh = plsc.VectorSubcoreMesh(core_axis_name="c", subcore_axis_name="s")
    cp = pltpu.CompilerParams()
    if "needs_layout_passes" in pltpu.CompilerParams.__dataclass_fields__:
        cp = dataclasses.replace(cp, needs_layout_passes=False)

    scratch = (
        [pltpu.VMEM((UNIT,), jnp.int32)] * DEPTH
        + [pltpu.VMEM((UNIT,), jnp.int32)] * DEPTH
        + [pltpu.VMEM((UNIT,), jnp.float32)] * DEPTH
        + [pltpu.VMEM((UNIT,), jnp.int32)] * DEPTH
        + [pltpu.VMEM((UNIT, D), jnp.float32)] * DEPTH
        + [pltpu.VMEM_SHARED((N_NODES, D), jnp.float32)]
        + [pltpu.SemaphoreType.DMA] * (3 * DEPTH + 1)
    )

    @functools.partial(
        pl.kernel,
        mesh=mesh,
        compiler_params=cp,
        out_type=jax.ShapeDtypeStruct((NC, N_NODES, D), jnp.float32),
        scratch_types=scratch,
    )
    def spmm(hw_hbm, row_hbm, col_hbm, val_hbm, out_hbm, *sc):
        colv = sc[0:DEPTH]
        rowv = sc[DEPTH:2 * DEPTH]
        valv = sc[2 * DEPTH:3 * DEPTH]
        rowS = sc[3 * DEPTH:4 * DEPTH]
        msgs = sc[4 * DEPTH:5 * DEPTH]
        acc = sc[5 * DEPTH]
        sem_i = sc[5 * DEPTH + 1:5 * DEPTH + 1 + DEPTH]
        sem_g = sc[5 * DEPTH + 1 + DEPTH:5 * DEPTH + 1 + 2 * DEPTH]
        sem_s = sc[5 * DEPTH + 1 + 2 * DEPTH:5 * DEPTH + 1 + 3 * DEPTH]
        sem_z = sc[5 * DEPTH + 1 + 3 * DEPTH]

        c = lax.axis_index("c")
        s = lax.axis_index("s")
        wid = s * NC + c

        def issue_idx(uu, b):
            base = (wid * n + uu) * UNIT
            pltpu.async_copy(col_hbm.at[pl.ds(base, UNIT)], colv[b], sem_i[b])
            pltpu.async_copy(row_hbm.at[pl.ds(base, UNIT)], rowv[b], sem_i[b])
            pltpu.async_copy(val_hbm.at[pl.ds(base, UNIT)], valv[b], sem_i[b])

        def wait_idx(b):
            src = col_hbm.at[pl.ds(0, UNIT)]
            pltpu.make_async_copy(src, colv[b], sem_i[b]).wait()
            pltpu.make_async_copy(src, rowv[b], sem_i[b]).wait()
            vsrc = val_hbm.at[pl.ds(0, UNIT)]
            pltpu.make_async_copy(vsrc, valv[b], sem_i[b]).wait()

        def issue_gather(b):
            pltpu.async_copy(hw_hbm.at[colv[b]], msgs[b], sem_g[b])

        def wait_gather(b):
            pltpu.make_async_copy(hw_hbm.at[colv[b]], msgs[b],
                                  sem_g[b]).wait()

        def issue_scatter(b):
            for j in range(UNIT // LANES):
                sl = pl.ds(j * LANES, LANES)
                rowS[b][sl] = rowv[b][sl]
            pltpu.async_copy(msgs[b], acc.at[rowS[b]], sem_s[b], add=True)

        def wait_scatter(b):
            pltpu.make_async_copy(msgs[b], acc.at[rowS[b]], sem_s[b]).wait()

        def scale(b):
            @pl.loop(0, UNIT, step=4)
            def _(e):
                vvs = [plsc.load_gather(
                    valv[b], [jnp.zeros((LANES,), jnp.int32) + (e + d)])
                    for d in range(4)]
                for j in range(D // LANES):
                    sl = pl.ds(j * LANES, LANES)
                    for d in range(4):
                        msgs[b][e + d, sl] = msgs[b][e + d, sl] * vvs[d]

        for b in range(DEPTH):
            issue_idx(b, b)

        zeros16 = jnp.zeros((LANES,), jnp.float32)

        @pl.loop(0, CHUNK_ROWS)
        def _(e):
            for j in range(D // LANES):
                msgs[DEPTH - 1][e, pl.ds(j * LANES, LANES)] = zeros16

        zbuf = msgs[DEPTH - 1].at[pl.ds(0, CHUNK_ROWS)]

        @pl.loop(0, (N_CHUNKS + NS - 1) // NS)
        def _(i):
            cid = s + i * NS

            @pl.when(cid < N_CHUNKS)
            def _():
                pltpu.async_copy(
                    zbuf, acc.at[pl.ds(cid * CHUNK_ROWS, CHUNK_ROWS)], sem_z)

        for b in range(DEPTH - 1):
            wait_idx(b)
            issue_gather(b)

        @pl.loop(0, (N_CHUNKS + NS - 1) // NS)
        def _(i):
            cid = s + i * NS

            @pl.when(cid < N_CHUNKS)
            def _():
                pltpu.make_async_copy(
                    zbuf, acc.at[pl.ds(cid * CHUNK_ROWS, CHUNK_ROWS)],
                    sem_z).wait()

        plsc.subcore_barrier()

        @pl.loop(0, n // DEPTH)
        def _(u):
            for b in range(DEPTH):
                uu = u * DEPTH + b
                bN = (b + DEPTH - 1) % DEPTH
                wait_gather(b)
                scale(b)
                issue_scatter(b)

                @pl.when(uu + DEPTH < n)
                def _():
                    issue_idx(uu + DEPTH, b)

                @pl.when(uu >= 1)
                def _():
                    wait_scatter(bN)

                @pl.when(uu + DEPTH - 1 < n)
                def _():
                    wait_idx(bN)
                    issue_gather(bN)

        wait_scatter((n - 1) % DEPTH)
        plsc.subcore_barrier()

        @pl.loop(0, (N_CHUNKS + NS - 1) // NS)
        def _(i):
            cid = s + i * NS

            @pl.when(cid < N_CHUNKS)
            def _():
                pltpu.async_copy(
                    acc.at[pl.ds(cid * CHUNK_ROWS, CHUNK_ROWS)],
                    out_hbm.at[c, pl.ds(cid * CHUNK_ROWS, CHUNK_ROWS)], sem_z)

        @pl.loop(0, (N_CHUNKS + NS - 1) // NS)
        def _(i):
            cid = s + i * NS

            @pl.when(cid < N_CHUNKS)
            def _():
                pltpu.make_async_copy(
                    acc.at[pl.ds(cid * CHUNK_ROWS, CHUNK_ROWS)],
                    out_hbm.at[c, pl.ds(cid * CHUNK_ROWS, CHUNK_ROWS)],
                    sem_z).wait()

    return spmm(HW, row, col, val)


def _finish_body(p_ref, w_ref, b_ref, o_ref):
    x = p_ref[0] + p_ref[1]
    y = jnp.dot(x, w_ref[...], preferred_element_type=jnp.float32)
    o_ref[...] = jnp.maximum(y + b_ref[...], 0.0)


def _finish(partials, W, bias):
    BM = 1000
    return pl.pallas_call(
        _finish_body,
        grid=(N_NODES // BM,),
        in_specs=[
            pl.BlockSpec((NC, BM, D), lambda i: (0, i, 0)),
            pl.BlockSpec((D, D), lambda i: (0, 0)),
            pl.BlockSpec((1, D), lambda i: (0, 0)),
        ],
        out_specs=pl.BlockSpec((BM, D), lambda i: (i, 0)),
        out_shape=jax.ShapeDtypeStruct((N_NODES, D), jnp.float32),
    )(partials, W, bias.reshape(1, D))


def kernel(A_edge_index, A_values, H, W, bias):
    row = A_edge_index[0]
    col = A_edge_index[1]
    E = row.shape[0]
    grain = NW * UNIT * DEPTH
    E_pad = ((E + grain - 1) // grain) * grain
    pad = E_pad - E
    if pad:
        spread = (jnp.arange(pad, dtype=jnp.int32) * 13) % N_NODES
        row = jnp.concatenate([row, spread])
        col = jnp.concatenate([col, spread])
        val = jnp.concatenate([A_values, jnp.zeros((pad,), jnp.float32)])
    else:
        val = A_values
    units_per_tile = E_pad // (NW * UNIT)

    partials = _spmm_partials(H, row, col, val, units_per_tile)
    return _finish(partials, W, bias)
